# Initial kernel scaffold; baseline (speedup 1.0000x reference)
import jax
import jax.numpy as jnp
from jax.experimental import pallas as pl


def kernel(x, table, W, b):
    raise NotImplementedError("write your pallas kernel here")


# speedup vs baseline: 2.8612x; 2.8612x over previous
import functools

import jax
import jax.numpy as jnp
from jax import lax
from jax.experimental import pallas as pl
from jax.experimental.pallas import tpu as pltpu
from jax.experimental.pallas import tpu_sc as plsc

B = 16384
H = 50
E = 32

NC = 2
NS = 16
NW = NC * NS
ROWS_PER_W = B // NW
CB = 32
NCHUNK = ROWS_PER_W // CB
CHUNK_I = CB * H


def _sc_pooled_sum(xflat, table):
    mesh = plsc.VectorSubcoreMesh(core_axis_name="c", subcore_axis_name="s")

    @functools.partial(
        pl.kernel,
        out_type=jax.ShapeDtypeStruct((B * E,), jnp.float32),
        mesh=mesh,
        compiler_params=pltpu.CompilerParams(use_tc_tiling_on_sc=False),
        scratch_types=[
            pltpu.VMEM((CHUNK_I,), jnp.int32),
            pltpu.VMEM((CHUNK_I,), jnp.int32),
            pltpu.VMEM((CHUNK_I, E), jnp.float32),
            pltpu.VMEM((CHUNK_I, E), jnp.float32),
            pltpu.VMEM((CB * E,), jnp.float32),
            pltpu.SemaphoreType.DMA,
            pltpu.SemaphoreType.DMA,
        ],
    )
    def body(x_hbm, table_hbm, out_hbm, idx0, idx1, rows0, rows1, stage,
             sem0, sem1):
        wid = lax.axis_index("s") * NC + lax.axis_index("c")
        ibase = wid * (ROWS_PER_W * H)
        obase = wid * (ROWS_PER_W * E)

        idx = (idx0, idx1)
        rows = (rows0, rows1)
        sems = (sem0, sem1)
        handles = [None, None]

        pltpu.sync_copy(x_hbm.at[pl.ds(ibase, CHUNK_I)], idx[0])
        handles[0] = pltpu.async_copy(table_hbm.at[idx[0]], rows[0], sems[0])

        for c in range(NCHUNK):
            cur = c % 2
            nxt = (c + 1) % 2
            if c + 1 < NCHUNK:
                pltpu.sync_copy(
                    x_hbm.at[pl.ds(ibase + (c + 1) * CHUNK_I, CHUNK_I)],
                    idx[nxt])
                handles[nxt] = pltpu.async_copy(
                    table_hbm.at[idx[nxt]], rows[nxt], sems[nxt])
            handles[cur].wait()
            rref = rows[cur]

            def row_body(bi, _, rref=rref):
                base = bi * H
                a0 = rref[base, 0:16]
                a1 = rref[base, 16:32]
                for j in range(1, H):
                    a0 = a0 + rref[base + j, 0:16]
                    a1 = a1 + rref[base + j, 16:32]
                stage[pl.ds(bi * E, 16)] = a0
                stage[pl.ds(bi * E + 16, 16)] = a1
                return 0

            lax.fori_loop(0, CB, row_body, 0)
            pltpu.sync_copy(
                stage, out_hbm.at[pl.ds(obase + c * (CB * E), CB * E)])

    return body(xflat, table)


def _tc_tail(pooled_sum, W, b2):
    BB = 2048

    def body(p_ref, w_ref, b_ref, o_ref):
        p = p_ref[...] * (1.0 / H)
        h = jnp.dot(p, w_ref[...], preferred_element_type=jnp.float32)
        h = jnp.maximum(h + b_ref[...], 0.0)
        m = jnp.max(h, axis=1, keepdims=True)
        e = jnp.exp(h - m)
        o_ref[...] = (h - m) - jnp.log(jnp.sum(e, axis=1, keepdims=True))

    return pl.pallas_call(
        body,
        grid=(B // BB,),
        in_specs=[
            pl.BlockSpec((BB, E), lambda i: (i, 0)),
            pl.BlockSpec((E, 2), lambda i: (0, 0)),
            pl.BlockSpec((1, 2), lambda i: (0, 0)),
        ],
        out_specs=pl.BlockSpec((BB, 2), lambda i: (i, 0)),
        out_shape=jax.ShapeDtypeStruct((B, 2), jnp.float32),
    )(pooled_sum, W, b2)


def kernel(x, table, W, b):
    xflat = x.astype(jnp.int32).reshape(B * H)
    pooled_sum = _sc_pooled_sum(xflat, table).reshape(B, E)
    return _tc_tail(pooled_sum, W, b.reshape(1, 2))
